# baseline (speedup 1.0000x reference)
import functools

import jax
import jax.numpy as jnp
from jax import lax
from jax.experimental import pallas as pl
from jax.experimental.pallas import tpu as pltpu
from jax.experimental.pallas import tpu_sc as plsc

N = 10000
E = 320000
D = 128

NC = 2
NS = 16
NW = NC * NS

CH = 128
NCH = 80
E_PAD = NW * NCH * CH
NP = 10240
RPT = NP // NS

_mesh = plsc.VectorSubcoreMesh(core_axis_name="c", subcore_axis_name="s")


@functools.partial(
    pl.kernel,
    out_type=jax.ShapeDtypeStruct((NC, NP, D), jnp.float32),
    mesh=_mesh,
    scratch_types=[
        pltpu.VMEM_SHARED((NP, D), jnp.float32),
        pltpu.VMEM((CH, D), jnp.float32),
        pltpu.VMEM((CH, D), jnp.float32),
        pltpu.VMEM((CH,), jnp.int32),
        pltpu.VMEM((CH,), jnp.int32),
        pltpu.VMEM((CH,), jnp.int32),
        pltpu.VMEM((CH,), jnp.int32),
        pltpu.VMEM((CH,), jnp.int32),
        pltpu.VMEM((CH,), jnp.int32),
        pltpu.VMEM((CH,), jnp.int32),
        pltpu.VMEM((CH,), jnp.int32),
        pltpu.SemaphoreType.DMA,
        pltpu.SemaphoreType.DMA,
        pltpu.SemaphoreType.DMA,
        pltpu.SemaphoreType.DMA,
        pltpu.SemaphoreType.DMA,
        pltpu.SemaphoreType.DMA,
        pltpu.SemaphoreType.DMA,
        pltpu.SemaphoreType.DMA,
        pltpu.SemaphoreType.DMA,
        pltpu.SemaphoreType.DMA,
        pltpu.SemaphoreType.DMA,
        pltpu.SemaphoreType.DMA,
    ],
)
def _spmm_sc(x_hbm, srcp_hbm, dstp_hbm, zeros_hbm, out_hbm,
             acc, r0b, r1b, s0, s1, s2, s3, d0, d1, d2, d3,
             g0, g1, i0, i1, i2, i3, k0, k1, k2, k3, t0, t1):
    rows = [r0b, r1b]
    sbuf = [s0, s1, s2, s3]
    dbuf = [d0, d1, d2, d3]
    gsem = [g0, g1]
    isem = [i0, i1, i2, i3]
    dsem = [k0, k1, k2, k3]
    ssem = [t0, t1]
    c = lax.axis_index("c")
    sid = lax.axis_index("s")
    w = sid * NC + c

    row0 = sid * RPT
    pltpu.sync_copy(zeros_hbm.at[pl.ds(row0, RPT)], acc.at[pl.ds(row0, RPT)])
    plsc.subcore_barrier()

    e0 = w * NCH * CH

    for j in range(4):
        pltpu.async_copy(srcp_hbm.at[pl.ds(e0 + j * CH, CH)], sbuf[j], isem[j])
        pltpu.async_copy(dstp_hbm.at[pl.ds(e0 + j * CH, CH)], dbuf[j], dsem[j])
    pltpu.make_async_copy(
        srcp_hbm.at[pl.ds(e0, CH)], sbuf[0], isem[0]).wait()
    pltpu.async_copy(x_hbm.at[sbuf[0]], rows[0], gsem[0])

    def group(g, _):
        for u in range(4):
            j = g * 4 + u
            b = u % 2
            pltpu.make_async_copy(x_hbm.at[sbuf[u]], rows[b], gsem[b]).wait()
            pltpu.make_async_copy(
                dstp_hbm.at[pl.ds(e0 + j * CH, CH)], dbuf[u], dsem[u]).wait()

            pltpu.async_copy(rows[b], acc.at[dbuf[u]], ssem[b], add=True)

            @pl.when(j + 1 < NCH)
            def _():
                uu = (u + 1) % 4
                up = (u + 3) % 4

                @pl.when(j >= 1)
                def _():
                    pltpu.make_async_copy(
                        rows[1 - b], acc.at[dbuf[up]], ssem[1 - b]).wait()

                pltpu.make_async_copy(
                    srcp_hbm.at[pl.ds(e0 + (j + 1) * CH, CH)],
                    sbuf[uu], isem[uu]).wait()
                pltpu.async_copy(x_hbm.at[sbuf[uu]], rows[1 - b], gsem[1 - b])

                @pl.when((j >= 1) & (j + 3 < NCH))
                def _():
                    pltpu.async_copy(
                        dstp_hbm.at[pl.ds(e0 + (j + 3) * CH, CH)],
                        dbuf[up], dsem[up])

            @pl.when(j + 4 < NCH)
            def _():
                pltpu.async_copy(
                    srcp_hbm.at[pl.ds(e0 + (j + 4) * CH, CH)], sbuf[u], isem[u])
        return 0

    lax.fori_loop(0, NCH // 4, group, 0)

    pltpu.make_async_copy(rows[0], acc.at[dbuf[2]], ssem[0]).wait()
    pltpu.make_async_copy(rows[1], acc.at[dbuf[3]], ssem[1]).wait()

    plsc.subcore_barrier()
    pltpu.sync_copy(acc.at[pl.ds(row0, RPT)], out_hbm.at[c, pl.ds(row0, RPT)])


BR = 1000


def _dense_body(p_ref, wt_ref, b_ref, a_ref, o_ref):
    neigh = p_ref[0] + p_ref[1]
    h = jnp.dot(neigh, wt_ref[...], preferred_element_type=jnp.float32)
    h = h + b_ref[...]
    a = a_ref[0, 0]
    h = jnp.where(h >= 0, h, a * h)
    m = jnp.max(h, axis=1, keepdims=True)
    e = jnp.exp(h - m)
    o_ref[...] = e / jnp.sum(e, axis=1, keepdims=True)


def kernel(x, edge_index, W, b, prelu_a):
    ei = edge_index.astype(jnp.int32)
    ppt = NCH * CH - E // NW
    i_pad = jnp.arange(ppt, dtype=jnp.int32)[None, :]
    w_pad = jnp.arange(NW, dtype=jnp.int32)[:, None]
    spad = (i_pad * 41 + w_pad * 313) % N
    dpad = N + (i_pad + w_pad * 8) % (NP - N)
    srcp = jnp.concatenate([ei[0].reshape(NW, E // NW), spad], axis=1).reshape(-1)
    dstp = jnp.concatenate([ei[1].reshape(NW, E // NW), dpad], axis=1).reshape(-1)
    zeros = jnp.zeros((NP, D), jnp.float32)

    parts = _spmm_sc(x, srcp, dstp, zeros)

    wt = W.T
    b2 = b.reshape(1, D)
    a2 = prelu_a.reshape(1, 1)

    out = pl.pallas_call(
        _dense_body,
        grid=(N // BR,),
        in_specs=[
            pl.BlockSpec((NC, BR, D), lambda i: (0, i, 0)),
            pl.BlockSpec((D, D), lambda i: (0, 0)),
            pl.BlockSpec((1, D), lambda i: (0, 0)),
            pl.BlockSpec(memory_space=pltpu.SMEM),
        ],
        out_specs=pl.BlockSpec((BR, D), lambda i: (i, 0)),
        out_shape=jax.ShapeDtypeStruct((N, D), jnp.float32),
    )(parts, wt, b2, a2)
    return out


# speedup vs baseline: 1.0096x; 1.0096x over previous
import functools

import jax
import jax.numpy as jnp
from jax import lax
from jax.experimental import pallas as pl
from jax.experimental.pallas import tpu as pltpu
from jax.experimental.pallas import tpu_sc as plsc

N = 10000
E = 320000
D = 128

NC = 2
NS = 16
NW = NC * NS

CH = 128
NCH = 80
E_PAD = NW * NCH * CH
NP = 10240
RPT = NP // NS

_mesh = plsc.VectorSubcoreMesh(core_axis_name="c", subcore_axis_name="s")


@functools.partial(
    pl.kernel,
    out_type=jax.ShapeDtypeStruct((NC, NP, D), jnp.float32),
    mesh=_mesh,
    scratch_types=[
        pltpu.VMEM_SHARED((NP, D), jnp.float32),
        pltpu.VMEM((CH, D), jnp.float32),
        pltpu.VMEM((CH, D), jnp.float32),
        pltpu.VMEM((CH,), jnp.int32),
        pltpu.VMEM((CH,), jnp.int32),
        pltpu.VMEM((CH,), jnp.int32),
        pltpu.VMEM((CH,), jnp.int32),
        pltpu.VMEM((CH,), jnp.int32),
        pltpu.VMEM((CH,), jnp.int32),
        pltpu.VMEM((CH,), jnp.int32),
        pltpu.VMEM((CH,), jnp.int32),
        pltpu.SemaphoreType.DMA,
        pltpu.SemaphoreType.DMA,
        pltpu.SemaphoreType.DMA,
        pltpu.SemaphoreType.DMA,
        pltpu.SemaphoreType.DMA,
        pltpu.SemaphoreType.DMA,
        pltpu.SemaphoreType.DMA,
        pltpu.SemaphoreType.DMA,
        pltpu.SemaphoreType.DMA,
        pltpu.SemaphoreType.DMA,
        pltpu.SemaphoreType.DMA,
        pltpu.SemaphoreType.DMA,
    ],
)
def _spmm_sc(x_hbm, srcp_hbm, dstp_hbm, zeros_hbm, out_hbm,
             acc, r0b, r1b, s0, s1, s2, s3, d0, d1, d2, d3,
             g0, g1, i0, i1, i2, i3, k0, k1, k2, k3, t0, t1):
    rows = [r0b, r1b]
    sbuf = [s0, s1, s2, s3]
    dbuf = [d0, d1, d2, d3]
    gsem = [g0, g1]
    isem = [i0, i1, i2, i3]
    dsem = [k0, k1, k2, k3]
    ssem = [t0, t1]
    c = lax.axis_index("c")
    sid = lax.axis_index("s")
    w = sid * NC + c

    row0 = sid * RPT
    pltpu.sync_copy(zeros_hbm.at[pl.ds(row0, RPT)], acc.at[pl.ds(row0, RPT)])
    plsc.subcore_barrier()

    e0 = w * NCH * CH

    for j in range(4):
        pltpu.async_copy(srcp_hbm.at[pl.ds(e0 + j * CH, CH)], sbuf[j], isem[j])
        pltpu.async_copy(dstp_hbm.at[pl.ds(e0 + j * CH, CH)], dbuf[j], dsem[j])
    pltpu.make_async_copy(
        srcp_hbm.at[pl.ds(e0, CH)], sbuf[0], isem[0]).wait()
    pltpu.async_copy(x_hbm.at[sbuf[0]], rows[0], gsem[0])

    def group(g, _):
        for u in range(4):
            j = g * 4 + u
            b = u % 2
            pltpu.make_async_copy(x_hbm.at[sbuf[u]], rows[b], gsem[b]).wait()
            pltpu.make_async_copy(
                dstp_hbm.at[pl.ds(e0 + j * CH, CH)], dbuf[u], dsem[u]).wait()

            pltpu.async_copy(rows[b], acc.at[dbuf[u]], ssem[b], add=True)

            @pl.when(j + 1 < NCH)
            def _():
                uu = (u + 1) % 4
                up = (u + 3) % 4

                @pl.when(j >= 1)
                def _():
                    pltpu.make_async_copy(
                        rows[1 - b], acc.at[dbuf[up]], ssem[1 - b]).wait()

                pltpu.make_async_copy(
                    srcp_hbm.at[pl.ds(e0 + (j + 1) * CH, CH)],
                    sbuf[uu], isem[uu]).wait()
                pltpu.async_copy(x_hbm.at[sbuf[uu]], rows[1 - b], gsem[1 - b])

                @pl.when((j >= 1) & (j + 3 < NCH))
                def _():
                    pltpu.async_copy(
                        dstp_hbm.at[pl.ds(e0 + (j + 3) * CH, CH)],
                        dbuf[up], dsem[up])

            @pl.when(j + 4 < NCH)
            def _():
                pltpu.async_copy(
                    srcp_hbm.at[pl.ds(e0 + (j + 4) * CH, CH)], sbuf[u], isem[u])
        return 0

    lax.fori_loop(0, NCH // 4, group, 0)

    pltpu.make_async_copy(rows[0], acc.at[dbuf[2]], ssem[0]).wait()
    pltpu.make_async_copy(rows[1], acc.at[dbuf[3]], ssem[1]).wait()

    plsc.subcore_barrier()
    pltpu.sync_copy(acc.at[pl.ds(row0, RPT)], out_hbm.at[c, pl.ds(row0, RPT)])


BR = 2000


def _dense_body(p_ref, wt_ref, b_ref, a_ref, o_ref):
    neigh = p_ref[0] + p_ref[1]
    h = jnp.dot(neigh, wt_ref[...], preferred_element_type=jnp.float32)
    h = h + b_ref[...]
    a = a_ref[0, 0]
    h = jnp.where(h >= 0, h, a * h)
    m = jnp.max(h, axis=1, keepdims=True)
    e = jnp.exp(h - m)
    o_ref[...] = e / jnp.sum(e, axis=1, keepdims=True)


def kernel(x, edge_index, W, b, prelu_a):
    ei = edge_index.astype(jnp.int32)
    ppt = NCH * CH - E // NW
    i_pad = jnp.arange(ppt, dtype=jnp.int32)[None, :]
    w_pad = jnp.arange(NW, dtype=jnp.int32)[:, None]
    spad = (i_pad * 41 + w_pad * 313) % N
    dpad = N + (i_pad + w_pad * 8) % (NP - N)
    srcp = jnp.concatenate([ei[0].reshape(NW, E // NW), spad], axis=1).reshape(-1)
    dstp = jnp.concatenate([ei[1].reshape(NW, E // NW), dpad], axis=1).reshape(-1)
    zeros = jnp.zeros((NP, D), jnp.float32)

    parts = _spmm_sc(x, srcp, dstp, zeros)

    wt = W.T
    b2 = b.reshape(1, D)
    a2 = prelu_a.reshape(1, 1)

    out = pl.pallas_call(
        _dense_body,
        grid=(N // BR,),
        in_specs=[
            pl.BlockSpec((NC, BR, D), lambda i: (0, i, 0)),
            pl.BlockSpec((D, D), lambda i: (0, 0)),
            pl.BlockSpec((1, D), lambda i: (0, 0)),
            pl.BlockSpec(memory_space=pltpu.SMEM),
        ],
        out_specs=pl.BlockSpec((BR, D), lambda i: (i, 0)),
        out_shape=jax.ShapeDtypeStruct((N, D), jnp.float32),
    )(parts, wt, b2, a2)
    return out
